# baseline (speedup 1.0000x reference)
import functools

import jax
import jax.numpy as jnp
from jax import lax
from jax.experimental import pallas as pl
from jax.experimental.pallas import tpu as pltpu
from jax.experimental.pallas import tpu_sc as plsc
from jax._src.pallas import mpmd

PRE_SEQ_LEN = 128
OUT_DIM = 12 * 2 * 768
BATCH = 32

NC = 2
NS = 16

SP_TOTAL = 10240
SP_COLS = SP_TOTAL // NC
TL_TOTAL = 6144
TL_COLS = TL_TOTAL // NS
SCS_TOTAL = OUT_DIM - SP_TOTAL - TL_TOTAL
SCS_COLS = SCS_TOTAL // NC
TL_BATCHES = BATCH // NC
BATCHES_PER_TILE = BATCH // NS
TAB_ROWS_PER_TILE = PRE_SEQ_LEN // NS
IDX_CHUNK_B = 8


def _scs_fn(prefix_hbm, table_hbm, out_hbm, spmem, tab_v, idx_v, smem_idx,
            sem_sp, sem_tl, sem_scs):
    c = lax.axis_index("c")
    col0 = SP_TOTAL + TL_TOTAL + c * SCS_COLS

    def outer(bb, carry):
        pltpu.sync_copy(prefix_hbm.at[pl.ds(bb * IDX_CHUNK_B, IDX_CHUNK_B), :],
                        smem_idx)

        def inner(r, carry2):
            batch = bb * IDX_CHUNK_B + r // PRE_SEQ_LEN
            row = r % PRE_SEQ_LEN
            ridx = smem_idx[r // PRE_SEQ_LEN, row]
            pltpu.async_copy(
                table_hbm.at[ridx, pl.ds(col0, SCS_COLS)],
                out_hbm.at[batch, row, pl.ds(col0, SCS_COLS)],
                sem_scs,
            )
            return carry2

        lax.fori_loop(0, IDX_CHUNK_B * PRE_SEQ_LEN, inner, 0)
        return carry

    lax.fori_loop(0, BATCH // IDX_CHUNK_B, outer, 0)

    def drain(r, carry):
        pltpu.make_async_copy(
            table_hbm.at[0, pl.ds(col0, SCS_COLS)],
            out_hbm.at[0, 0, pl.ds(col0, SCS_COLS)],
            sem_scs,
        ).wait()
        return carry

    lax.fori_loop(0, BATCH * PRE_SEQ_LEN, drain, 0)


def _tec_fn(prefix_hbm, table_hbm, out_hbm, spmem, tab_v, idx_v, smem_idx,
            sem_sp, sem_tl, sem_scs):
    c = lax.axis_index("c")
    s = lax.axis_index("s")
    sp0 = c * SP_COLS
    tl0 = SP_TOTAL + s * TL_COLS
    b0 = c * TL_BATCHES

    pltpu.sync_copy(prefix_hbm, idx_v)
    pltpu.sync_copy(table_hbm.at[:, pl.ds(tl0, TL_COLS)], tab_v)

    def issue_tl(k, carry):
        batch = b0 + k // (PRE_SEQ_LEN // 16)
        row0 = (k % (PRE_SEQ_LEN // 16)) * 16
        v = idx_v[batch, pl.ds(row0, 16)]
        for j in range(16):
            pltpu.async_copy(
                tab_v.at[v[j]],
                out_hbm.at[batch, row0 + j, pl.ds(tl0, TL_COLS)],
                sem_tl,
            )
        return carry

    n_tl = TL_BATCHES * PRE_SEQ_LEN // 16
    lax.fori_loop(0, n_tl, issue_tl, 0)

    tr0 = s * TAB_ROWS_PER_TILE
    pltpu.sync_copy(
        table_hbm.at[pl.ds(tr0, TAB_ROWS_PER_TILE), pl.ds(sp0, SP_COLS)],
        spmem.at[pl.ds(tr0, TAB_ROWS_PER_TILE), :],
    )
    plsc.subcore_barrier()

    def issue_sp(k, carry):
        r0 = k * 16
        batch = s * BATCHES_PER_TILE + r0 // PRE_SEQ_LEN
        row0 = r0 % PRE_SEQ_LEN
        v = idx_v[batch, pl.ds(row0, 16)]
        for j in range(16):
            pltpu.async_copy(
                spmem.at[v[j]],
                out_hbm.at[batch, row0 + j, pl.ds(sp0, SP_COLS)],
                sem_sp,
            )
        return carry

    n_sp = BATCHES_PER_TILE * PRE_SEQ_LEN // 16
    lax.fori_loop(0, n_sp, issue_sp, 0)

    def drain_tl(k, carry):
        pltpu.make_async_copy(
            tab_v.at[0],
            out_hbm.at[b0, 0, pl.ds(tl0, TL_COLS)],
            sem_tl,
        ).wait()
        return carry

    lax.fori_loop(0, n_tl * 16, drain_tl, 0)

    def drain_sp(k, carry):
        pltpu.make_async_copy(
            spmem.at[0],
            out_hbm.at[s * BATCHES_PER_TILE, 0, pl.ds(sp0, SP_COLS)],
            sem_sp,
        ).wait()
        return carry

    lax.fori_loop(0, n_sp * 16, drain_sp, 0)


_scs_mesh = plsc.ScalarSubcoreMesh(axis_name="c", num_cores=NC)
_tec_mesh = plsc.VectorSubcoreMesh(core_axis_name="c", subcore_axis_name="s")

_gather = mpmd.mpmd_map(
    [
        (_scs_mesh, _scs_fn),
        (_tec_mesh, _tec_fn),
    ],
    out_types=jax.ShapeDtypeStruct((BATCH, PRE_SEQ_LEN, OUT_DIM), jnp.float32),
    scratch_types=[
        pltpu.VMEM_SHARED((PRE_SEQ_LEN, SP_COLS), jnp.float32),
        (pltpu.VMEM @ _tec_mesh)((PRE_SEQ_LEN, TL_COLS), jnp.float32),
        (pltpu.VMEM @ _tec_mesh)((BATCH, PRE_SEQ_LEN), jnp.int32),
        (pltpu.SMEM @ _scs_mesh)((IDX_CHUNK_B, PRE_SEQ_LEN), jnp.int32),
        pltpu.SemaphoreType.DMA @ _tec_mesh,
        pltpu.SemaphoreType.DMA @ _tec_mesh,
        pltpu.SemaphoreType.DMA @ _scs_mesh,
    ],
    compiler_params=pltpu.CompilerParams(use_tc_tiling_on_sc=True),
)


@jax.jit
def kernel(prefix, emb_table):
    return _gather(prefix.astype(jnp.int32), emb_table)


# speedup vs baseline: 6.3693x; 6.3693x over previous
import functools

import jax
import jax.numpy as jnp
from jax import lax
from jax.experimental import pallas as pl
from jax.experimental.pallas import tpu as pltpu
from jax.experimental.pallas import tpu_sc as plsc

PRE_SEQ_LEN = 128
OUT_DIM = 12 * 2 * 768
BATCH = 32

NC = 2
NS = 16

SP_TOTAL = 12288
SP_COLS = SP_TOTAL // NC
TL_COLS = (OUT_DIM - SP_TOTAL) // NS
TL_BATCHES = BATCH // NC
BATCHES_PER_TILE = BATCH // NS
TAB_ROWS_PER_TILE = PRE_SEQ_LEN // NS


def _body(prefix_hbm, table_hbm, out_hbm, spmem, tab_v, idx_v, sem_sp, sem_sp2,
          sem_tl):
    c = lax.axis_index("c")
    s = lax.axis_index("s")
    sp0 = c * SP_COLS
    tl0 = SP_TOTAL + s * TL_COLS
    b0 = c * TL_BATCHES

    pltpu.sync_copy(prefix_hbm, idx_v)
    pltpu.sync_copy(table_hbm.at[:, pl.ds(tl0, TL_COLS)], tab_v)

    def issue_tl(k, carry):
        batch = b0 + k // (PRE_SEQ_LEN // 16)
        row0 = (k % (PRE_SEQ_LEN // 16)) * 16
        v = idx_v[batch, pl.ds(row0, 16)]
        for j in range(16):
            pltpu.async_copy(
                tab_v.at[v[j]],
                out_hbm.at[batch, row0 + j, pl.ds(tl0, TL_COLS)],
                sem_tl,
            )
        return carry

    n_tl = TL_BATCHES * PRE_SEQ_LEN // 16
    lax.fori_loop(0, n_tl, issue_tl, 0)

    tr0 = s * TAB_ROWS_PER_TILE
    pltpu.sync_copy(
        table_hbm.at[pl.ds(tr0, TAB_ROWS_PER_TILE), pl.ds(sp0, SP_COLS)],
        spmem.at[pl.ds(tr0, TAB_ROWS_PER_TILE), :],
    )
    plsc.subcore_barrier()

    def make_issue_sp(bb, sem):
        def issue_sp(k, carry):
            batch = s * BATCHES_PER_TILE + bb
            row0 = k * 16
            v = idx_v[batch, pl.ds(row0, 16)]
            for j in range(16):
                pltpu.async_copy(
                    spmem.at[v[j]],
                    out_hbm.at[batch, row0 + j, pl.ds(sp0, SP_COLS)],
                    sem,
                )
            return carry
        return issue_sp

    n_sp = PRE_SEQ_LEN // 16
    lax.fori_loop(0, n_sp, make_issue_sp(0, sem_sp), 0)
    lax.fori_loop(0, n_sp, make_issue_sp(1, sem_sp2), 0)

    def drain_tl(k, carry):
        pltpu.make_async_copy(
            tab_v.at[0],
            out_hbm.at[b0, 0, pl.ds(tl0, TL_COLS)],
            sem_tl,
        ).wait()
        return carry

    lax.fori_loop(0, n_tl * 16, drain_tl, 0)

    def make_drain_sp(sem):
        def drain_sp(k, carry):
            pltpu.make_async_copy(
                spmem.at[0],
                out_hbm.at[s * BATCHES_PER_TILE, 0, pl.ds(sp0, SP_COLS)],
                sem,
            ).wait()
            return carry
        return drain_sp

    lax.fori_loop(0, n_sp * 16, make_drain_sp(sem_sp), 0)
    lax.fori_loop(0, n_sp * 16, make_drain_sp(sem_sp2), 0)


_gather = functools.partial(
    pl.kernel,
    out_type=jax.ShapeDtypeStruct((BATCH, PRE_SEQ_LEN, OUT_DIM), jnp.float32),
    mesh=plsc.VectorSubcoreMesh(core_axis_name="c", subcore_axis_name="s"),
    scratch_types=[
        pltpu.VMEM_SHARED((PRE_SEQ_LEN, SP_COLS), jnp.float32),
        pltpu.VMEM((PRE_SEQ_LEN, TL_COLS), jnp.float32),
        pltpu.VMEM((BATCH, PRE_SEQ_LEN), jnp.int32),
        pltpu.SemaphoreType.DMA,
        pltpu.SemaphoreType.DMA,
        pltpu.SemaphoreType.DMA,
    ],
    compiler_params=pltpu.CompilerParams(use_tc_tiling_on_sc=True),
)(_body)


@jax.jit
def kernel(prefix, emb_table):
    return _gather(prefix.astype(jnp.int32), emb_table)


# speedup vs baseline: 6.4530x; 1.0131x over previous
import functools

import jax
import jax.numpy as jnp
from jax import lax
from jax.experimental import pallas as pl
from jax.experimental.pallas import tpu as pltpu
from jax.experimental.pallas import tpu_sc as plsc

PRE_SEQ_LEN = 128
OUT_DIM = 12 * 2 * 768
BATCH = 32

NC = 2
NS = 16

SP_TOTAL = 12288
SP_COLS = SP_TOTAL // NC
TL_COLS = (OUT_DIM - SP_TOTAL) // NS
TL_BATCHES = BATCH // NC
BATCHES_PER_TILE = BATCH // NS
TAB_ROWS_PER_TILE = PRE_SEQ_LEN // NS


def _body(prefix_hbm, table_hbm, out_hbm, spmem, tab_v, idx_v, sem_sp, sem_tl,
          sem_stage):
    c = lax.axis_index("c")
    s = lax.axis_index("s")
    sp0 = c * SP_COLS
    tl0 = SP_TOTAL + s * TL_COLS
    b0 = c * TL_BATCHES

    tr0 = s * TAB_ROWS_PER_TILE
    stage = pltpu.async_copy(
        table_hbm.at[pl.ds(tr0, TAB_ROWS_PER_TILE), pl.ds(sp0, SP_COLS)],
        spmem.at[pl.ds(tr0, TAB_ROWS_PER_TILE), :],
        sem_stage,
    )

    pltpu.sync_copy(prefix_hbm, idx_v)
    pltpu.sync_copy(table_hbm.at[:, pl.ds(tl0, TL_COLS)], tab_v)

    def issue_tl(k, carry):
        batch = b0 + k // (PRE_SEQ_LEN // 16)
        row0 = (k % (PRE_SEQ_LEN // 16)) * 16
        v = idx_v[batch, pl.ds(row0, 16)]
        for j in range(16):
            pltpu.async_copy(
                tab_v.at[v[j]],
                out_hbm.at[batch, row0 + j, pl.ds(tl0, TL_COLS)],
                sem_tl,
            )
        return carry

    n_tl = TL_BATCHES * PRE_SEQ_LEN // 16
    lax.fori_loop(0, n_tl, issue_tl, 0)

    stage.wait()
    plsc.subcore_barrier()

    def issue_sp(k, carry):
        r0 = k * 16
        batch = s * BATCHES_PER_TILE + r0 // PRE_SEQ_LEN
        row0 = r0 % PRE_SEQ_LEN
        v = idx_v[batch, pl.ds(row0, 16)]
        for j in range(16):
            pltpu.async_copy(
                spmem.at[v[j]],
                out_hbm.at[batch, row0 + j, pl.ds(sp0, SP_COLS)],
                sem_sp,
            )
        return carry

    n_sp = BATCHES_PER_TILE * PRE_SEQ_LEN // 16
    lax.fori_loop(0, n_sp, issue_sp, 0)

    def drain_tl(k, carry):
        pltpu.make_async_copy(
            tab_v.at[0],
            out_hbm.at[b0, 0, pl.ds(tl0, TL_COLS)],
            sem_tl,
        ).wait()
        return carry

    lax.fori_loop(0, n_tl * 16, drain_tl, 0)

    def drain_sp(k, carry):
        pltpu.make_async_copy(
            spmem.at[0],
            out_hbm.at[s * BATCHES_PER_TILE, 0, pl.ds(sp0, SP_COLS)],
            sem_sp,
        ).wait()
        return carry

    lax.fori_loop(0, n_sp * 16, drain_sp, 0)


_gather = functools.partial(
    pl.kernel,
    out_type=jax.ShapeDtypeStruct((BATCH, PRE_SEQ_LEN, OUT_DIM), jnp.float32),
    mesh=plsc.VectorSubcoreMesh(core_axis_name="c", subcore_axis_name="s"),
    scratch_types=[
        pltpu.VMEM_SHARED((PRE_SEQ_LEN, SP_COLS), jnp.float32),
        pltpu.VMEM((PRE_SEQ_LEN, TL_COLS), jnp.float32),
        pltpu.VMEM((BATCH, PRE_SEQ_LEN), jnp.int32),
        pltpu.SemaphoreType.DMA,
        pltpu.SemaphoreType.DMA,
        pltpu.SemaphoreType.DMA,
    ],
    compiler_params=pltpu.CompilerParams(use_tc_tiling_on_sc=True),
)(_body)


@jax.jit
def kernel(prefix, emb_table):
    return _gather(prefix.astype(jnp.int32), emb_table)
